# baseline (speedup 1.0000x reference)
import functools

import jax
import jax.numpy as jnp
from jax.experimental import pallas as pl
from jax.experimental.pallas import tpu as pltpu
from jax.experimental.pallas import tpu_sc as plsc

_HIGH = jax.lax.Precision.HIGHEST


def _pick_div(n, target):
    best = 1
    for d in range(1, target + 1):
        if n % d == 0:
            best = d
    return best


def _prep_body(y_ref, fy_ref, w1a_ref, w1b_ref, b1_ref, af_ref, b_ref):
    y = y_ref[...]
    a = jax.lax.dot_general(y, w1a_ref[...], (((1,), (0,)), ((), ())),
                            preferred_element_type=jnp.float32,
                            precision=_HIGH)
    af_ref[...] = jnp.concatenate([a, fy_ref[...]], axis=-1)
    b = jax.lax.dot_general(y, w1b_ref[...], (((1,), (0,)), ((), ())),
                            preferred_element_type=jnp.float32,
                            precision=_HIGH) + b1_ref[...]
    pad = b_ref.shape[1] - b.shape[1]
    b_ref[...] = jnp.concatenate(
        [b, jnp.zeros((b.shape[0], pad), jnp.float32)], axis=-1)


def _prep(y, f_y, w1a, w1b, b1):
    n, dc = y.shape
    d = f_y.shape[1]
    h = w1a.shape[1]
    bn = _pick_div(n, 1000)
    return pl.pallas_call(
        _prep_body,
        grid=(n // bn,),
        in_specs=[
            pl.BlockSpec((bn, dc), lambda i: (i, 0)),
            pl.BlockSpec((bn, d), lambda i: (i, 0)),
            pl.BlockSpec((dc, h), lambda i: (0, 0)),
            pl.BlockSpec((dc, h), lambda i: (0, 0)),
            pl.BlockSpec((1, h), lambda i: (0, 0)),
        ],
        out_specs=[
            pl.BlockSpec((bn, h + d), lambda i: (i, 0)),
            pl.BlockSpec((bn, 128), lambda i: (i, 0)),
        ],
        out_shape=[
            jax.ShapeDtypeStruct((n, h + d), jnp.float32),
            jax.ShapeDtypeStruct((n, 128), jnp.float32),
        ],
    )(y, f_y, w1a, w1b, b1)


def _sc_gather(table, idx, chunk=800):
    orig = idx.shape[0]
    d = table.shape[1]
    info = plsc.get_sparse_core_info()
    nw = info.num_cores * info.num_subcores
    per_w = -(-orig // (nw * chunk)) * chunk
    tot = per_w * nw
    if tot != orig:
        idx = jnp.concatenate(
            [idx, jnp.zeros((tot - orig,), idx.dtype)])
    niter = per_w // chunk
    mesh = plsc.VectorSubcoreMesh(core_axis_name="c", subcore_axis_name="s")

    @functools.partial(
        pl.kernel,
        out_type=jax.ShapeDtypeStruct((tot, d), table.dtype),
        mesh=mesh,
        scratch_types=[
            pltpu.VMEM((chunk,), jnp.int32),
            pltpu.VMEM((chunk, d), table.dtype),
            pltpu.SemaphoreType.DMA,
        ],
    )
    def gather_kernel(x_hbm, i_hbm, o_hbm, idx_v, rows_v, sem):
        wid = (jax.lax.axis_index("s") * info.num_cores
               + jax.lax.axis_index("c"))
        base0 = wid * per_w

        @pl.loop(0, niter)
        def _(j):
            base = base0 + j * chunk
            pltpu.sync_copy(i_hbm.at[pl.ds(base, chunk)], idx_v)
            pltpu.async_copy(x_hbm.at[idx_v], rows_v, sem).wait()
            pltpu.sync_copy(rows_v, o_hbm.at[pl.ds(base, chunk)])

    out = gather_kernel(table, idx)
    return out[:orig] if tot != orig else out


def _edge_body(nc, nt, h_dim, d, g_ref, g2_ref, w2_ref, b2_ref, p_ref,
               carry_ref):
    @pl.when(pl.program_id(0) == 0)
    def _():
        carry_ref[...] = jnp.zeros_like(carry_ref)

    g = g_ref[...]
    a = g[:, :h_dim]
    f = g[:, h_dim:]
    x = a + g2_ref[:, :h_dim]
    hh = x * 0.5 * (1.0 + jax.lax.erf(x * 0.7071067811865476))
    k = jax.lax.dot_general(hh, w2_ref[...], (((1,), (0,)), ((), ())),
                            preferred_element_type=jnp.float32,
                            precision=_HIGH) + b2_ref[...]
    r = k * f

    r3 = r.reshape(nc, nt, d)
    it = jax.lax.broadcasted_iota(jnp.int32, (nt, nt), 0)
    jt = jax.lax.broadcasted_iota(jnp.int32, (nt, nt), 1)
    tri = (it >= jt).astype(jnp.float32)
    ic = jax.lax.broadcasted_iota(jnp.int32, (nc, nc), 0)
    jc = jax.lax.broadcasted_iota(jnp.int32, (nc, nc), 1)
    tris = (ic > jc).astype(jnp.float32)

    tri_b = jnp.broadcast_to(tri, (nc, nt, nt))
    cum = jax.lax.dot_general(tri_b, r3, (((2,), (1,)), ((0,), (0,))),
                              preferred_element_type=jnp.float32,
                              precision=_HIGH)
    tot = jnp.sum(r3, axis=1)
    pre = jax.lax.dot_general(tris, tot, (((1,), (0,)), ((), ())),
                              preferred_element_type=jnp.float32,
                              precision=_HIGH)
    p = cum + pre[:, None, :] + carry_ref[...][None, :, :]
    p2 = p.reshape(nc * nt, d)
    p_ref[...] = jnp.concatenate(
        [p2, jnp.zeros((nc * nt, p_ref.shape[1] - d), jnp.float32)], axis=-1)
    carry_ref[...] = p2[nc * nt - 1:nc * nt, :]


def _edge_stage(g, g2, w2, b2):
    e = g.shape[0]
    h = w2.shape[0]
    d = g.shape[1] - h
    bl = _pick_div(e, 1600)
    nt = _pick_div(bl, 40)
    nc = bl // nt
    body = functools.partial(_edge_body, nc, nt, h, d)
    return pl.pallas_call(
        body,
        grid=(e // bl,),
        in_specs=[
            pl.BlockSpec((bl, h + d), lambda i: (i, 0)),
            pl.BlockSpec((bl, 128), lambda i: (i, 0)),
            pl.BlockSpec((h, d), lambda i: (0, 0)),
            pl.BlockSpec((1, d), lambda i: (0, 0)),
        ],
        out_specs=pl.BlockSpec((bl, 128), lambda i: (i, 0)),
        out_shape=jax.ShapeDtypeStruct((e, 128), jnp.float32),
        scratch_shapes=[pltpu.VMEM((1, d), jnp.float32)],
    )(g, g2, w2, b2)


def _combine_body(d, hi_ref, lo_ref, shi_ref, slo_ref, o_ref):
    o_ref[...] = (hi_ref[:, :d] * shi_ref[...]
                  - lo_ref[:, :d] * slo_ref[...])


def _combine(hi_g, lo_g, s_hi, s_lo, d):
    n = hi_g.shape[0]
    w = hi_g.shape[1]
    bn = _pick_div(n, 1000)
    return pl.pallas_call(
        functools.partial(_combine_body, d),
        grid=(n // bn,),
        in_specs=[
            pl.BlockSpec((bn, w), lambda i: (i, 0)),
            pl.BlockSpec((bn, w), lambda i: (i, 0)),
            pl.BlockSpec((bn, 1), lambda i: (i, 0)),
            pl.BlockSpec((bn, 1), lambda i: (i, 0)),
        ],
        out_specs=pl.BlockSpec((bn, d), lambda i: (i, 0)),
        out_shape=jax.ShapeDtypeStruct((n, d), jnp.float32),
    )(hi_g, lo_g, s_hi, s_lo)


def kernel(y, neighbors_index, neighbors_row_splits, f_y, W1, b1, W2, b2):
    n, dc = y.shape
    e = neighbors_index.shape[0]
    d = f_y.shape[1]
    h = W1.shape[1]

    rs = neighbors_row_splits.astype(jnp.int32)
    nbr = neighbors_index.astype(jnp.int32)

    seg = (jnp.searchsorted(rs, jnp.arange(e, dtype=jnp.int32), side="right")
           .astype(jnp.int32) - 1)

    w1a = W1[:dc]
    w1b = W1[dc:]
    b1r = b1.reshape(1, h)
    b2r = b2.reshape(1, d)

    af, b_tab = _prep(y, f_y, w1a, w1b, b1r)

    g = _sc_gather(af, nbr)
    g2 = _sc_gather(b_tab, seg)

    p = _edge_stage(g, g2, W2, b2r)

    lo = jnp.maximum(rs[:-1] - 1, 0)
    hi = jnp.maximum(rs[1:] - 1, 0)
    bidx = jnp.concatenate([lo, hi]).astype(jnp.int32)
    pg = _sc_gather(p, bidx)
    lo_g = pg[:n]
    hi_g = pg[n:]

    cnt = rs[1:] - rs[:-1]
    inv = 1.0 / jnp.maximum(cnt, 1).astype(jnp.float32)
    s_lo = ((rs[:-1] > 0).astype(jnp.float32) * inv).reshape(n, 1)
    s_hi = ((rs[1:] > 0).astype(jnp.float32) * inv).reshape(n, 1)

    return _combine(hi_g, lo_g, s_hi, s_lo, d)


# speedup vs baseline: 15.4888x; 15.4888x over previous
import functools

import jax
import jax.numpy as jnp
from jax.experimental import pallas as pl
from jax.experimental.pallas import tpu as pltpu
from jax.experimental.pallas import tpu_sc as plsc

_HIGH = jax.lax.Precision.HIGHEST


def _pick_div(n, target):
    best = 1
    for d in range(1, target + 1):
        if n % d == 0:
            best = d
    return best


def _prep_body(y_ref, fy_ref, w1a_ref, w1b_ref, b1_ref, af_ref, b_ref):
    y = y_ref[...]
    a = jax.lax.dot_general(y, w1a_ref[...], (((1,), (0,)), ((), ())),
                            preferred_element_type=jnp.float32,
                            precision=_HIGH)
    af_ref[...] = jnp.concatenate([a, fy_ref[...]], axis=-1)
    b = jax.lax.dot_general(y, w1b_ref[...], (((1,), (0,)), ((), ())),
                            preferred_element_type=jnp.float32,
                            precision=_HIGH) + b1_ref[...]
    pad = b_ref.shape[1] - b.shape[1]
    b_ref[...] = jnp.concatenate(
        [b, jnp.zeros((b.shape[0], pad), jnp.float32)], axis=-1)


def _prep(y, f_y, w1a, w1b, b1):
    n, dc = y.shape
    d = f_y.shape[1]
    h = w1a.shape[1]
    bn = _pick_div(n, 1000)
    return pl.pallas_call(
        _prep_body,
        grid=(n // bn,),
        in_specs=[
            pl.BlockSpec((bn, dc), lambda i: (i, 0)),
            pl.BlockSpec((bn, d), lambda i: (i, 0)),
            pl.BlockSpec((dc, h), lambda i: (0, 0)),
            pl.BlockSpec((dc, h), lambda i: (0, 0)),
            pl.BlockSpec((1, h), lambda i: (0, 0)),
        ],
        out_specs=[
            pl.BlockSpec((bn, h + d), lambda i: (i, 0)),
            pl.BlockSpec((bn, 128), lambda i: (i, 0)),
        ],
        out_shape=[
            jax.ShapeDtypeStruct((n, h + d), jnp.float32),
            jax.ShapeDtypeStruct((n, 128), jnp.float32),
        ],
    )(y, f_y, w1a, w1b, b1)


def _sc_gather(table, idx, chunk=800):
    orig = idx.shape[0]
    d = table.shape[1]
    info = plsc.get_sparse_core_info()
    nw = info.num_cores * info.num_subcores
    per_w = -(-orig // (nw * chunk)) * chunk
    tot = per_w * nw
    if tot != orig:
        idx = jnp.concatenate(
            [idx, jnp.zeros((tot - orig,), idx.dtype)])
    niter = per_w // chunk
    mesh = plsc.VectorSubcoreMesh(core_axis_name="c", subcore_axis_name="s")

    @functools.partial(
        pl.kernel,
        out_type=jax.ShapeDtypeStruct((tot, d), table.dtype),
        mesh=mesh,
        scratch_types=[
            pltpu.VMEM((chunk,), jnp.int32),
            pltpu.VMEM((chunk, d), table.dtype),
            pltpu.SemaphoreType.DMA,
        ],
    )
    def gather_kernel(x_hbm, i_hbm, o_hbm, idx_v, rows_v, sem):
        wid = (jax.lax.axis_index("s") * info.num_cores
               + jax.lax.axis_index("c"))
        base0 = wid * per_w

        @pl.loop(0, niter)
        def _(j):
            base = base0 + j * chunk
            pltpu.sync_copy(i_hbm.at[pl.ds(base, chunk)], idx_v)
            pltpu.async_copy(x_hbm.at[idx_v], rows_v, sem).wait()
            pltpu.sync_copy(rows_v, o_hbm.at[pl.ds(base, chunk)])

    out = gather_kernel(table, idx)
    return out[:orig] if tot != orig else out


def _edge_body(nc, nt, h_dim, d, g_ref, g2_ref, w2_ref, b2_ref, p_ref,
               carry_ref):
    @pl.when(pl.program_id(0) == 0)
    def _():
        carry_ref[...] = jnp.zeros_like(carry_ref)

    g = g_ref[...]
    a = g[:, :h_dim]
    f = g[:, h_dim:]
    x = a + g2_ref[:, :h_dim]
    hh = x * 0.5 * (1.0 + jax.lax.erf(x * 0.7071067811865476))
    k = jax.lax.dot_general(hh, w2_ref[...], (((1,), (0,)), ((), ())),
                            preferred_element_type=jnp.float32,
                            precision=_HIGH) + b2_ref[...]
    r = k * f

    r3 = r.reshape(nc, nt, d)
    it = jax.lax.broadcasted_iota(jnp.int32, (nt, nt), 0)
    jt = jax.lax.broadcasted_iota(jnp.int32, (nt, nt), 1)
    tri = (it >= jt).astype(jnp.float32)
    ic = jax.lax.broadcasted_iota(jnp.int32, (nc, nc), 0)
    jc = jax.lax.broadcasted_iota(jnp.int32, (nc, nc), 1)
    tris = (ic > jc).astype(jnp.float32)

    tri_b = jnp.broadcast_to(tri, (nc, nt, nt))
    cum = jax.lax.dot_general(tri_b, r3, (((2,), (1,)), ((0,), (0,))),
                              preferred_element_type=jnp.float32,
                              precision=_HIGH)
    tot = jnp.sum(r3, axis=1)
    pre = jax.lax.dot_general(tris, tot, (((1,), (0,)), ((), ())),
                              preferred_element_type=jnp.float32,
                              precision=_HIGH)
    p = cum + pre[:, None, :] + carry_ref[...][None, :, :]
    p2 = p.reshape(nc * nt, d)
    p_ref[...] = jnp.concatenate(
        [p2, jnp.zeros((nc * nt, p_ref.shape[1] - d), jnp.float32)], axis=-1)
    carry_ref[...] = p2[nc * nt - 1:nc * nt, :]


def _edge_stage(g, g2, w2, b2):
    e = g.shape[0]
    h = w2.shape[0]
    d = g.shape[1] - h
    bl = _pick_div(e, 1600)
    nt = _pick_div(bl, 40)
    nc = bl // nt
    body = functools.partial(_edge_body, nc, nt, h, d)
    return pl.pallas_call(
        body,
        grid=(e // bl,),
        in_specs=[
            pl.BlockSpec((bl, h + d), lambda i: (i, 0)),
            pl.BlockSpec((bl, 128), lambda i: (i, 0)),
            pl.BlockSpec((h, d), lambda i: (0, 0)),
            pl.BlockSpec((1, d), lambda i: (0, 0)),
        ],
        out_specs=pl.BlockSpec((bl, 128), lambda i: (i, 0)),
        out_shape=jax.ShapeDtypeStruct((e, 128), jnp.float32),
        scratch_shapes=[pltpu.VMEM((1, d), jnp.float32)],
    )(g, g2, w2, b2)


def _combine_body(d, hi_ref, lo_ref, shi_ref, slo_ref, o_ref):
    o_ref[...] = (hi_ref[:, :d] * shi_ref[...]
                  - lo_ref[:, :d] * slo_ref[...])


def _combine(hi_g, lo_g, s_hi, s_lo, d):
    n = hi_g.shape[0]
    w = hi_g.shape[1]
    bn = _pick_div(n, 1000)
    return pl.pallas_call(
        functools.partial(_combine_body, d),
        grid=(n // bn,),
        in_specs=[
            pl.BlockSpec((bn, w), lambda i: (i, 0)),
            pl.BlockSpec((bn, w), lambda i: (i, 0)),
            pl.BlockSpec((bn, 1), lambda i: (i, 0)),
            pl.BlockSpec((bn, 1), lambda i: (i, 0)),
        ],
        out_specs=pl.BlockSpec((bn, d), lambda i: (i, 0)),
        out_shape=jax.ShapeDtypeStruct((n, d), jnp.float32),
    )(hi_g, lo_g, s_hi, s_lo)


def kernel(y, neighbors_index, neighbors_row_splits, f_y, W1, b1, W2, b2):
    n, dc = y.shape
    e = neighbors_index.shape[0]
    d = f_y.shape[1]
    h = W1.shape[1]

    rs = neighbors_row_splits.astype(jnp.int32)
    nbr = neighbors_index.astype(jnp.int32)

    z = jnp.zeros((e,), jnp.int32).at[rs[1:-1]].add(1)
    seg = jnp.cumsum(z)

    w1a = W1[:dc]
    w1b = W1[dc:]
    b1r = b1.reshape(1, h)
    b2r = b2.reshape(1, d)

    af, b_tab = _prep(y, f_y, w1a, w1b, b1r)

    g = _sc_gather(af, nbr)
    g2 = _sc_gather(b_tab, seg)

    p = _edge_stage(g, g2, W2, b2r)

    lo = jnp.maximum(rs[:-1] - 1, 0)
    hi = jnp.maximum(rs[1:] - 1, 0)
    bidx = jnp.concatenate([lo, hi]).astype(jnp.int32)
    pg = _sc_gather(p, bidx)
    lo_g = pg[:n]
    hi_g = pg[n:]

    cnt = rs[1:] - rs[:-1]
    inv = 1.0 / jnp.maximum(cnt, 1).astype(jnp.float32)
    s_lo = ((rs[:-1] > 0).astype(jnp.float32) * inv).reshape(n, 1)
    s_hi = ((rs[1:] > 0).astype(jnp.float32) * inv).reshape(n, 1)

    return _combine(hi_g, lo_g, s_hi, s_lo, d)


# speedup vs baseline: 16.6939x; 1.0778x over previous
import functools

import jax
import jax.numpy as jnp
from jax.experimental import pallas as pl
from jax.experimental.pallas import tpu as pltpu
from jax.experimental.pallas import tpu_sc as plsc

_HIGH = jax.lax.Precision.HIGHEST


def _pick_div(n, target):
    best = 1
    for d in range(1, target + 1):
        if n % d == 0:
            best = d
    return best


def _prep_body(y_ref, fy_ref, w1a_ref, w1b_ref, b1_ref, af_ref, b_ref):
    y = y_ref[...]
    a = jax.lax.dot_general(y, w1a_ref[...], (((1,), (0,)), ((), ())),
                            preferred_element_type=jnp.float32,
                            precision=_HIGH)
    af_ref[...] = jnp.concatenate([a, fy_ref[...]], axis=-1)
    b = jax.lax.dot_general(y, w1b_ref[...], (((1,), (0,)), ((), ())),
                            preferred_element_type=jnp.float32,
                            precision=_HIGH) + b1_ref[...]
    pad = b_ref.shape[1] - b.shape[1]
    b_ref[...] = jnp.concatenate(
        [b, jnp.zeros((b.shape[0], pad), jnp.float32)], axis=-1)


def _prep(y, f_y, w1a, w1b, b1):
    n, dc = y.shape
    d = f_y.shape[1]
    h = w1a.shape[1]
    bn = _pick_div(n, 1000)
    return pl.pallas_call(
        _prep_body,
        grid=(n // bn,),
        in_specs=[
            pl.BlockSpec((bn, dc), lambda i: (i, 0)),
            pl.BlockSpec((bn, d), lambda i: (i, 0)),
            pl.BlockSpec((dc, h), lambda i: (0, 0)),
            pl.BlockSpec((dc, h), lambda i: (0, 0)),
            pl.BlockSpec((1, h), lambda i: (0, 0)),
        ],
        out_specs=[
            pl.BlockSpec((bn, h + d), lambda i: (i, 0)),
            pl.BlockSpec((bn, 128), lambda i: (i, 0)),
        ],
        out_shape=[
            jax.ShapeDtypeStruct((n, h + d), jnp.float32),
            jax.ShapeDtypeStruct((n, 128), jnp.float32),
        ],
    )(y, f_y, w1a, w1b, b1)


def _sc_gather(table, idx, chunk=800, trim=True):
    orig = idx.shape[0]
    d = table.shape[1]
    info = plsc.get_sparse_core_info()
    nw = info.num_cores * info.num_subcores
    per_w = -(-orig // (nw * chunk)) * chunk
    tot = per_w * nw
    if tot != orig:
        idx = jnp.concatenate(
            [idx, jnp.zeros((tot - orig,), idx.dtype)])
    niter = per_w // chunk
    mesh = plsc.VectorSubcoreMesh(core_axis_name="c", subcore_axis_name="s")

    @functools.partial(
        pl.kernel,
        out_type=jax.ShapeDtypeStruct((tot, d), table.dtype),
        mesh=mesh,
        scratch_types=[
            pltpu.VMEM((chunk,), jnp.int32),
            pltpu.VMEM((chunk, d), table.dtype),
            pltpu.SemaphoreType.DMA,
        ],
    )
    def gather_kernel(x_hbm, i_hbm, o_hbm, idx_v, rows_v, sem):
        wid = (jax.lax.axis_index("s") * info.num_cores
               + jax.lax.axis_index("c"))
        base0 = wid * per_w

        @pl.loop(0, niter)
        def _(j):
            base = base0 + j * chunk
            pltpu.sync_copy(i_hbm.at[pl.ds(base, chunk)], idx_v)
            pltpu.async_copy(x_hbm.at[idx_v], rows_v, sem).wait()
            pltpu.sync_copy(rows_v, o_hbm.at[pl.ds(base, chunk)])

    out = gather_kernel(table, idx)
    if trim and tot != orig:
        return out[:orig]
    return out


def _edge_body(nc, nt, h_dim, d, g_ref, g2_ref, w2_ref, b2_ref, p_ref,
               carry_ref):
    @pl.when(pl.program_id(0) == 0)
    def _():
        carry_ref[...] = jnp.zeros_like(carry_ref)

    g = g_ref[...]
    a = g[:, :h_dim]
    f = g[:, h_dim:]
    x = a + g2_ref[:, :h_dim]
    hh = x * 0.5 * (1.0 + jax.lax.erf(x * 0.7071067811865476))
    k = jax.lax.dot_general(hh, w2_ref[...], (((1,), (0,)), ((), ())),
                            preferred_element_type=jnp.float32,
                            precision=_HIGH) + b2_ref[...]
    r = k * f

    r3 = r.reshape(nc, nt, d)
    it = jax.lax.broadcasted_iota(jnp.int32, (nt, nt), 0)
    jt = jax.lax.broadcasted_iota(jnp.int32, (nt, nt), 1)
    tri = (it >= jt).astype(jnp.float32)
    ic = jax.lax.broadcasted_iota(jnp.int32, (nc, nc), 0)
    jc = jax.lax.broadcasted_iota(jnp.int32, (nc, nc), 1)
    tris = (ic > jc).astype(jnp.float32)

    tri_b = jnp.broadcast_to(tri, (nc, nt, nt))
    cum = jax.lax.dot_general(tri_b, r3, (((2,), (1,)), ((0,), (0,))),
                              preferred_element_type=jnp.float32,
                              precision=_HIGH)
    tot = jnp.sum(r3, axis=1)
    pre = jax.lax.dot_general(tris, tot, (((1,), (0,)), ((), ())),
                              preferred_element_type=jnp.float32,
                              precision=_HIGH)
    p = cum + pre[:, None, :] + carry_ref[...][None, :, :]
    p2 = p.reshape(nc * nt, d)
    p_ref[...] = jnp.concatenate(
        [p2, jnp.zeros((nc * nt, p_ref.shape[1] - d), jnp.float32)], axis=-1)
    carry_ref[...] = p2[nc * nt - 1:nc * nt, :]


def _edge_stage(g, g2, w2, b2):
    e = g.shape[0]
    h = w2.shape[0]
    d = g.shape[1] - h
    bl = _pick_div(e, 1600)
    nt = _pick_div(bl, 40)
    nc = bl // nt
    body = functools.partial(_edge_body, nc, nt, h, d)
    return pl.pallas_call(
        body,
        grid=(e // bl,),
        in_specs=[
            pl.BlockSpec((bl, h + d), lambda i: (i, 0)),
            pl.BlockSpec((bl, 128), lambda i: (i, 0)),
            pl.BlockSpec((h, d), lambda i: (0, 0)),
            pl.BlockSpec((1, d), lambda i: (0, 0)),
        ],
        out_specs=pl.BlockSpec((bl, 128), lambda i: (i, 0)),
        out_shape=jax.ShapeDtypeStruct((e, 128), jnp.float32),
        scratch_shapes=[pltpu.VMEM((1, d), jnp.float32)],
    )(g, g2, w2, b2)


def _combine_body(d, hi_ref, lo_ref, shi_ref, slo_ref, o_ref):
    o_ref[...] = (hi_ref[:, :d] * shi_ref[...]
                  - lo_ref[:, :d] * slo_ref[...])


def _combine(hi_g, lo_g, s_hi, s_lo, d):
    n = hi_g.shape[0]
    w = hi_g.shape[1]
    bn = _pick_div(n, 1000)
    return pl.pallas_call(
        functools.partial(_combine_body, d),
        grid=(n // bn,),
        in_specs=[
            pl.BlockSpec((bn, w), lambda i: (i, 0)),
            pl.BlockSpec((bn, w), lambda i: (i, 0)),
            pl.BlockSpec((bn, 1), lambda i: (i, 0)),
            pl.BlockSpec((bn, 1), lambda i: (i, 0)),
        ],
        out_specs=pl.BlockSpec((bn, d), lambda i: (i, 0)),
        out_shape=jax.ShapeDtypeStruct((n, d), jnp.float32),
    )(hi_g, lo_g, s_hi, s_lo)


def kernel(y, neighbors_index, neighbors_row_splits, f_y, W1, b1, W2, b2):
    n, dc = y.shape
    e = neighbors_index.shape[0]
    d = f_y.shape[1]
    h = W1.shape[1]

    rs = neighbors_row_splits.astype(jnp.int32)
    nbr = neighbors_index.astype(jnp.int32)

    z = jnp.zeros((e,), jnp.int32).at[rs[1:-1]].add(1)
    seg = jnp.cumsum(z)

    w1a = W1[:dc]
    w1b = W1[dc:]
    b1r = b1.reshape(1, h)
    b2r = b2.reshape(1, d)

    af, b_tab = _prep(y, f_y, w1a, w1b, b1r)

    g = _sc_gather(af, nbr, trim=False)
    g2 = _sc_gather(b_tab, seg, trim=False)

    p = _edge_stage(g, g2, W2, b2r)

    lo = jnp.maximum(rs[:-1] - 1, 0)
    hi = jnp.maximum(rs[1:] - 1, 0)
    bidx = jnp.concatenate([lo, hi]).astype(jnp.int32)
    pg = _sc_gather(p, bidx)
    lo_g = pg[:n]
    hi_g = pg[n:]

    cnt = rs[1:] - rs[:-1]
    inv = 1.0 / jnp.maximum(cnt, 1).astype(jnp.float32)
    s_lo = ((rs[:-1] > 0).astype(jnp.float32) * inv).reshape(n, 1)
    s_hi = ((rs[1:] > 0).astype(jnp.float32) * inv).reshape(n, 1)

    return _combine(hi_g, lo_g, s_hi, s_lo, d)
